# baseline (speedup 1.0000x reference)
import jax
import jax.numpy as jnp
from jax import lax
from jax.experimental import pallas as pl
from jax.experimental.pallas import tpu as pltpu
from jax.experimental.pallas import tpu_sc as plsc

_B, _L = 1024, 50
_V, _De, _H, _P = 100000, 128, 1024, 512
_N = _B * _L
_L0 = 10
_L1 = _L - _L0
_TBL = 5
_BT = _TBL * _B

_NC, _NS = 2, 16
_NW = _NC * _NS


def _sc_gather_body(n_rows, ids_hbm, table_hbm, out_hbm, idx_v, rows_v, sem):
    rows_per_w = n_rows // _NW
    n_chunks = -(-rows_per_w // 800)
    chunk = rows_per_w // n_chunks
    wid = lax.axis_index("s") * _NC + lax.axis_index("c")
    base = wid * rows_per_w
    for c in range(n_chunks):
        off = base + c * chunk
        pltpu.sync_copy(ids_hbm.at[pl.ds(off, chunk)], idx_v)
        pltpu.async_copy(table_hbm.at[idx_v], rows_v, sem).wait()
        pltpu.sync_copy(rows_v, out_hbm.at[pl.ds(off, chunk)])


def _sc_gather(ids_chunk, table, n_rows):
    mesh = plsc.VectorSubcoreMesh(core_axis_name="c", subcore_axis_name="s")
    rows_per_w = n_rows // _NW
    chunk = rows_per_w // (-(-rows_per_w // 800))
    return pl.kernel(
        lambda *a: _sc_gather_body(n_rows, *a),
        out_type=jax.ShapeDtypeStruct((n_rows, _De), jnp.float32),
        mesh=mesh,
        scratch_types=[
            pltpu.VMEM((chunk,), jnp.int32),
            pltpu.VMEM((chunk, _De), jnp.float32),
            pltpu.SemaphoreType.DMA,
        ],
    )(ids_chunk, table)


def _tc_body(pos_ref, tt_ref, e_ref, w_ref, pt_ref, ty_ref, g_ref, b_ref, o_ref):
    for j in range(_TBL):
        proj = jnp.dot(e_ref[pl.ds(j * _B, _B), :].astype(jnp.bfloat16),
                       w_ref[...], preferred_element_type=jnp.float32)
        pos = pos_ref[j, 0, :]
        oh = (lax.broadcasted_iota(jnp.int32, (_B, _P), 1) == pos[:, None])
        p = jnp.dot(oh.astype(jnp.bfloat16), pt_ref[...],
                    preferred_element_type=jnp.float32)
        tt = tt_ref[j, 0, :].astype(jnp.float32)[:, None]
        t = ty_ref[0:1, :] + tt * (ty_ref[1:2, :] - ty_ref[0:1, :])
        x = proj + p + t
        mu = jnp.mean(x, axis=-1, keepdims=True)
        xc = x - mu
        var = jnp.mean(xc * xc, axis=-1, keepdims=True)
        xn = xc * lax.rsqrt(var + 1e-12)
        o_ref[pl.ds(j, 1)] = (xn * g_ref[...] + b_ref[...]).reshape(1, _B, _H)


def _tc_body_alias(prev_ref, pos_ref, tt_ref, e_ref, w_ref, pt_ref, ty_ref,
                   g_ref, b_ref, o_ref):
    del prev_ref
    _tc_body(pos_ref, tt_ref, e_ref, w_ref, pt_ref, ty_ref, g_ref, b_ref, o_ref)


_DATA_SPECS = [
    pl.BlockSpec((_TBL, 1, _B), lambda i: (i, 0, 0)),
    pl.BlockSpec((_TBL, 1, _B), lambda i: (i, 0, 0)),
    pl.BlockSpec((_BT, _De), lambda i: (i, 0)),
    pl.BlockSpec((_De, _H), lambda i: (0, 0)),
    pl.BlockSpec((_P, _H), lambda i: (0, 0)),
    pl.BlockSpec((2, _H), lambda i: (0, 0)),
    pl.BlockSpec((1, _H), lambda i: (0, 0)),
    pl.BlockSpec((1, _H), lambda i: (0, 0)),
]


def _tc_chunk0(pos_blocks, tt_blocks, e_rows, *tables):
    return pl.pallas_call(
        _tc_body,
        grid=(_L0 // _TBL,),
        in_specs=_DATA_SPECS,
        out_specs=pl.BlockSpec((_TBL, _B, _H), lambda i: (i, 0, 0)),
        out_shape=jax.ShapeDtypeStruct((_L, _B, _H), jnp.float32),
    )(pos_blocks, tt_blocks, e_rows, *tables)


def _tc_chunk1(prev, pos_blocks, tt_blocks, e_rows, *tables):
    return pl.pallas_call(
        _tc_body_alias,
        grid=(_L1 // _TBL,),
        in_specs=[pl.BlockSpec(memory_space=pl.ANY)] + _DATA_SPECS,
        out_specs=pl.BlockSpec((_TBL, _B, _H), lambda i: (i + _L0 // _TBL, 0, 0)),
        out_shape=jax.ShapeDtypeStruct((_L, _B, _H), jnp.float32),
        input_output_aliases={0: 0},
    )(prev, pos_blocks, tt_blocks, e_rows, *tables)


def kernel(entity_ids, position_ids, token_type_ids, entity_table, dense_w,
           position_table, type_table, ln_gamma, ln_beta):
    n0 = _L0 * _B
    ids_lb = entity_ids.T.reshape(_N)
    e0 = _sc_gather(ids_lb[:n0], entity_table, n0)
    e1 = _sc_gather(ids_lb[n0:], entity_table, _N - n0)
    pos_lb = position_ids.T.reshape(_L, 1, _B)
    tt_lb = token_type_ids.T.reshape(_L, 1, _B)
    tables = (dense_w.astype(jnp.bfloat16), position_table.astype(jnp.bfloat16),
              type_table, ln_gamma.reshape(1, _H), ln_beta.reshape(1, _H))
    chunk0 = _tc_chunk0(pos_lb[:_L0], tt_lb[:_L0], e0, *tables)
    out_lb = _tc_chunk1(chunk0, pos_lb[_L0:], tt_lb[_L0:], e1, *tables)
    return jnp.transpose(out_lb, (1, 0, 2))


# speedup vs baseline: 1.0160x; 1.0160x over previous
import jax
import jax.numpy as jnp
from jax import lax
from jax.experimental import pallas as pl
from jax.experimental.pallas import tpu as pltpu
from jax.experimental.pallas import tpu_sc as plsc

_B, _L = 1024, 50
_V, _De, _H, _P = 100000, 128, 1024, 512
_N = _B * _L
_L0 = 8
_L1 = _L - _L0
_TBL = 2
_BT = _TBL * _B

_NC, _NS = 2, 16
_NW = _NC * _NS


def _sc_gather_body(n_rows, ids_hbm, table_hbm, out_hbm, idx_v, rows_v, sem):
    rows_per_w = n_rows // _NW
    n_chunks = -(-rows_per_w // 800)
    chunk = rows_per_w // n_chunks
    wid = lax.axis_index("s") * _NC + lax.axis_index("c")
    base = wid * rows_per_w
    for c in range(n_chunks):
        off = base + c * chunk
        pltpu.sync_copy(ids_hbm.at[pl.ds(off, chunk)], idx_v)
        pltpu.async_copy(table_hbm.at[idx_v], rows_v, sem).wait()
        pltpu.sync_copy(rows_v, out_hbm.at[pl.ds(off, chunk)])


def _sc_gather(ids_chunk, table, n_rows):
    mesh = plsc.VectorSubcoreMesh(core_axis_name="c", subcore_axis_name="s")
    rows_per_w = n_rows // _NW
    chunk = rows_per_w // (-(-rows_per_w // 800))
    return pl.kernel(
        lambda *a: _sc_gather_body(n_rows, *a),
        out_type=jax.ShapeDtypeStruct((n_rows, _De), jnp.float32),
        mesh=mesh,
        scratch_types=[
            pltpu.VMEM((chunk,), jnp.int32),
            pltpu.VMEM((chunk, _De), jnp.float32),
            pltpu.SemaphoreType.DMA,
        ],
    )(ids_chunk, table)


def _tc_body(pos_ref, tt_ref, e_ref, w_ref, pt_ref, ty_ref, g_ref, b_ref,
             o_ref):
    for j in range(_TBL):
        proj = jnp.dot(e_ref[pl.ds(j * _B, _B), :].astype(jnp.bfloat16),
                       w_ref[...], preferred_element_type=jnp.float32)
        pos = pos_ref[j, 0, :]
        oh = (lax.broadcasted_iota(jnp.int32, (_B, _P), 1) == pos[:, None])
        p = jnp.dot(oh.astype(jnp.bfloat16), pt_ref[...],
                    preferred_element_type=jnp.float32)
        tt = tt_ref[j, 0, :].astype(jnp.float32)[:, None]
        t = ty_ref[0:1, :] + tt * (ty_ref[1:2, :] - ty_ref[0:1, :])
        x = proj + p + t
        mu = jnp.mean(x, axis=-1, keepdims=True)
        xc = x - mu
        var = jnp.mean(xc * xc, axis=-1, keepdims=True)
        xn = xc * lax.rsqrt(var + 1e-12)
        o_ref[pl.ds(j, 1)] = (xn * g_ref[...] + b_ref[...]).reshape(1, _B, _H)


def _tc_body_alias(prev_ref, pos_ref, tt_ref, e_ref, w_ref, pt_ref, ty_ref,
                   g_ref, b_ref, o_ref):
    del prev_ref
    _tc_body(pos_ref, tt_ref, e_ref, w_ref, pt_ref, ty_ref, g_ref, b_ref,
             o_ref)


_DATA_SPECS = [
    pl.BlockSpec((_TBL, 1, _B), lambda i: (i, 0, 0)),
    pl.BlockSpec((_TBL, 1, _B), lambda i: (i, 0, 0)),
    pl.BlockSpec((_BT, _De), lambda i: (i, 0)),
    pl.BlockSpec((_De, _H), lambda i: (0, 0)),
    pl.BlockSpec((_P, _H), lambda i: (0, 0)),
    pl.BlockSpec((2, _H), lambda i: (0, 0)),
    pl.BlockSpec((1, _H), lambda i: (0, 0)),
    pl.BlockSpec((1, _H), lambda i: (0, 0)),
]


def _tc_chunk0(pos_blocks, tt_blocks, e_rows, *tables):
    return pl.pallas_call(
        _tc_body,
        grid=(_L0 // _TBL,),
        in_specs=_DATA_SPECS,
        out_specs=pl.BlockSpec((_TBL, _B, _H), lambda i: (i, 0, 0)),
        out_shape=jax.ShapeDtypeStruct((_L, _B, _H), jnp.float32),
    )(pos_blocks, tt_blocks, e_rows, *tables)


def _tc_chunk1(prev, pos_blocks, tt_blocks, e_rows, *tables):
    return pl.pallas_call(
        _tc_body_alias,
        grid=(_L1 // _TBL,),
        in_specs=[pl.BlockSpec(memory_space=pl.ANY)] + _DATA_SPECS,
        out_specs=pl.BlockSpec((_TBL, _B, _H), lambda i: (i + _L0 // _TBL, 0, 0)),
        out_shape=jax.ShapeDtypeStruct((_L, _B, _H), jnp.float32),
        input_output_aliases={0: 0},
    )(prev, pos_blocks, tt_blocks, e_rows, *tables)


def kernel(entity_ids, position_ids, token_type_ids, entity_table, dense_w,
           position_table, type_table, ln_gamma, ln_beta):
    n0 = _L0 * _B
    ids_lb = entity_ids.T.reshape(_N)
    e0 = _sc_gather(ids_lb[:n0], entity_table, n0)
    e1 = _sc_gather(ids_lb[n0:], entity_table, _N - n0)
    pos_lb = position_ids.T.reshape(_L, 1, _B)
    tt_lb = token_type_ids.T.reshape(_L, 1, _B)

    tables = (dense_w.astype(jnp.bfloat16), position_table.astype(jnp.bfloat16),
              type_table, ln_gamma.reshape(1, _H), ln_beta.reshape(1, _H))
    chunk0 = _tc_chunk0(pos_lb[:_L0], tt_lb[:_L0], e0, *tables)
    out_lb = _tc_chunk1(chunk0, pos_lb[_L0:], tt_lb[_L0:], e1, *tables)
    return jnp.transpose(out_lb, (1, 0, 2))
